# baseline (speedup 1.0000x reference)
import functools

import jax
import jax.numpy as jnp
from jax import lax
from jax.experimental import pallas as pl
from jax.experimental.pallas import tpu as pltpu
from jax.experimental.pallas import tpu_sc as plsc

NC = 2
NS = 16
L = 16
CH = 128


def _sc_mesh():
    return plsc.VectorSubcoreMesh(core_axis_name="c", subcore_axis_name="s")


def _make_bincount(nw, nb, na):
    rpt = na // NS
    bpt = nw // NS

    @functools.partial(
        pl.kernel,
        out_type=jax.ShapeDtypeStruct((2, na), jnp.float32),
        mesh=_sc_mesh(),
        compiler_params=pltpu.CompilerParams(needs_layout_passes=False),
        scratch_types=[
            pltpu.VMEM((nb, CH), jnp.int32),
            pltpu.VMEM((na,), jnp.float32),
            pltpu.VMEM((NS, rpt), jnp.float32),
            pltpu.VMEM((rpt,), jnp.float32),
            pltpu.VMEM_SHARED((NS, na), jnp.float32),
        ],
    )
    def bincount(edges_hbm, out_hbm, idx_v, hist_v, red_v, deg_v, shared_h):
        c = lax.axis_index("c")
        s = lax.axis_index("s")
        zeros = jnp.zeros((L,), jnp.float32)
        ones = jnp.ones((L,), jnp.float32)

        def zero_body(i, _):
            hist_v[pl.ds(i * L, L)] = zeros
            return 0
        lax.fori_loop(0, na // L, zero_body, 0)

        for bi in range(bpt):
            pltpu.sync_copy(edges_hbm.at[c, s * bpt + bi], idx_v)

            def acc_body(j, _):
                for k in range(CH // L):
                    idx16 = idx_v[j, pl.ds(k * L, L)]
                    plsc.addupdate_scatter(hist_v, [idx16], ones)
                return 0
            lax.fori_loop(0, nb, acc_body, 0)

        pltpu.sync_copy(hist_v, shared_h.at[s])
        plsc.subcore_barrier()

        for t in range(NS):
            pltpu.sync_copy(shared_h.at[t, pl.ds(s * rpt, rpt)], red_v.at[t])

        def red_body(i, _):
            v = red_v[0, pl.ds(i * L, L)]
            for t in range(1, NS):
                v = v + red_v[t, pl.ds(i * L, L)]
            deg_v[pl.ds(i * L, L)] = v
            return 0
        lax.fori_loop(0, rpt // L, red_body, 0)

        pltpu.sync_copy(deg_v, out_hbm.at[c, pl.ds(s * rpt, rpt)])

    return bincount


def _make_msgpass(nw, nb, na, d):
    rpt = na // NS

    @functools.partial(
        pl.kernel,
        out_type=jax.ShapeDtypeStruct((2, na, d), jnp.float32),
        mesh=_sc_mesh(),
        scratch_types=[
            pltpu.VMEM((nb, CH), jnp.int32),
            pltpu.VMEM((nb, CH), jnp.int32),
            pltpu.VMEM((CH, d), jnp.float32),
            pltpu.VMEM_SHARED((na, d), jnp.float32),
            pltpu.SemaphoreType.DMA,
        ],
    )
    def msgpass(zs_hbm, edges_hbm, out_hbm, src_v, dst_v, rows_v, acc_sh, gsem):
        c = lax.axis_index("c")
        s = lax.axis_index("s")
        w = c * NS + s
        zeros = jnp.zeros((L,), jnp.float32)

        pltpu.sync_copy(edges_hbm.at[0, w], src_v)
        pltpu.sync_copy(edges_hbm.at[1, w], dst_v)

        def zrow(i, _):
            for k in range(d // L):
                rows_v[i, pl.ds(k * L, L)] = zeros
            return 0
        lax.fori_loop(0, CH, zrow, 0)
        for q in range(rpt // CH):
            pltpu.sync_copy(rows_v, acc_sh.at[pl.ds(s * rpt + q * CH, CH)])
        plsc.subcore_barrier()

        def edge_body(j, _):
            pltpu.async_copy(zs_hbm.at[src_v.at[j]], rows_v, gsem).wait()
            pltpu.sync_copy(rows_v, acc_sh.at[dst_v.at[j]], add=True)
            return 0
        lax.fori_loop(0, nb, edge_body, 0)

        plsc.subcore_barrier()
        pltpu.sync_copy(acc_sh.at[pl.ds(s * rpt, rpt)],
                        out_hbm.at[c, pl.ds(s * rpt, rpt)])

    return msgpass


def _tc_matmul_body(x_ref, w_ref, o_ref):
    o_ref[...] = jnp.dot(x_ref[...], w_ref[...],
                         preferred_element_type=jnp.float32)


def _make_tc_scale_body(blk, npad):
    del blk, npad
    def body(y_ref, d_ref, o_ref):
        o_ref[...] = y_ref[...] * lax.rsqrt(jnp.maximum(d_ref[...], 1.0))
    return body


def _tc_combine_body(p_ref, d_ref, b_ref, o_ref):
    nd = lax.rsqrt(jnp.maximum(d_ref[...], 1.0))
    o_ref[...] = (p_ref[0] + p_ref[1]) * nd + b_ref[...]


def kernel(features, edge_index, W, b):
    n, d_in = features.shape
    d_out = W.shape[1]
    e = edge_index.shape[1]

    nw = NC * NS
    rpt = (-(-n // NS) + CH - 1) // CH * CH
    na = NS * rpt
    ept = -(-e // (nw * 4 * CH)) * 4 * CH
    nb = ept // CH
    e_pad = nw * ept

    npad = e_pad - e
    pad2 = jnp.full((2, npad), n, jnp.int32)
    edges = jnp.concatenate([edge_index, pad2], axis=1).reshape(2, nw, nb, CH)

    degs = _make_bincount(nw, nb, na)(edges)
    rows_blk = 2000
    y = pl.pallas_call(
        _tc_matmul_body,
        grid=(n // rows_blk,),
        in_specs=[
            pl.BlockSpec((rows_blk, d_in), lambda i: (i, 0)),
            pl.BlockSpec((d_in, d_out), lambda i: (0, 0)),
        ],
        out_specs=pl.BlockSpec((rows_blk, d_out), lambda i: (i, 0)),
        out_shape=jax.ShapeDtypeStruct((n, d_out), jnp.float32),
    )(features, W)

    ds_col = degs[0, :n].reshape(n, 1)
    zs = pl.pallas_call(
        _make_tc_scale_body(rows_blk, npad),
        grid=(n // rows_blk,),
        in_specs=[
            pl.BlockSpec((rows_blk, d_out), lambda i: (i, 0)),
            pl.BlockSpec((rows_blk, 1), lambda i: (i, 0)),
        ],
        out_specs=pl.BlockSpec((rows_blk, d_out), lambda i: (i, 0)),
        out_shape=jax.ShapeDtypeStruct((na, d_out), jnp.float32),
    )(y, ds_col)

    parts = _make_msgpass(nw, nb, na, d_out)(zs, edges)

    dd_col = degs[1].reshape(na, 1)
    b_row = b.reshape(1, d_out)
    out_blk = 1000
    out = pl.pallas_call(
        _tc_combine_body,
        grid=(n // out_blk,),
        in_specs=[
            pl.BlockSpec((2, out_blk, d_out), lambda i: (0, i, 0)),
            pl.BlockSpec((out_blk, 1), lambda i: (i, 0)),
            pl.BlockSpec((1, d_out), lambda i: (0, 0)),
        ],
        out_specs=pl.BlockSpec((out_blk, d_out), lambda i: (i, 0)),
        out_shape=jax.ShapeDtypeStruct((n, d_out), jnp.float32),
    )(parts, dd_col, b_row)

    return out


# speedup vs baseline: 1.1321x; 1.1321x over previous
import functools

import jax
import jax.numpy as jnp
from jax import lax
from jax.experimental import pallas as pl
from jax.experimental.pallas import tpu as pltpu
from jax.experimental.pallas import tpu_sc as plsc

NC = 2
NS = 16
L = 16
CH = 128


def _sc_mesh():
    return plsc.VectorSubcoreMesh(core_axis_name="c", subcore_axis_name="s")


def _make_bincount(nw, nb, na):
    rpt = na // NS
    bpt = nw // NS

    @functools.partial(
        pl.kernel,
        out_type=jax.ShapeDtypeStruct((2, na), jnp.float32),
        mesh=_sc_mesh(),
        compiler_params=pltpu.CompilerParams(needs_layout_passes=False),
        scratch_types=[
            pltpu.VMEM((nb, CH), jnp.int32),
            pltpu.VMEM((na,), jnp.float32),
            pltpu.VMEM((NS, rpt), jnp.float32),
            pltpu.VMEM((rpt,), jnp.float32),
            pltpu.VMEM_SHARED((NS, na), jnp.float32),
        ],
    )
    def bincount(edges_hbm, out_hbm, idx_v, hist_v, red_v, deg_v, shared_h):
        c = lax.axis_index("c")
        s = lax.axis_index("s")
        zeros = jnp.zeros((L,), jnp.float32)
        ones = jnp.ones((L,), jnp.float32)

        def zero_body(i, _):
            hist_v[pl.ds(i * L, L)] = zeros
            return 0
        lax.fori_loop(0, na // L, zero_body, 0)

        for bi in range(bpt):
            pltpu.sync_copy(edges_hbm.at[c, s * bpt + bi], idx_v)

            def acc_body(j, _):
                for k in range(CH // L):
                    idx16 = idx_v[j, pl.ds(k * L, L)]
                    plsc.addupdate_scatter(hist_v, [idx16], ones)
                return 0
            lax.fori_loop(0, nb, acc_body, 0)

        pltpu.sync_copy(hist_v, shared_h.at[s])
        plsc.subcore_barrier()

        for t in range(NS):
            pltpu.sync_copy(shared_h.at[t, pl.ds(s * rpt, rpt)], red_v.at[t])

        def red_body(i, _):
            v = red_v[0, pl.ds(i * L, L)]
            for t in range(1, NS):
                v = v + red_v[t, pl.ds(i * L, L)]
            deg_v[pl.ds(i * L, L)] = v
            return 0
        lax.fori_loop(0, rpt // L, red_body, 0)

        pltpu.sync_copy(deg_v, out_hbm.at[c, pl.ds(s * rpt, rpt)])

    return bincount


def _make_msgpass(nw, nb, na, d):
    rpt = na // NS

    @functools.partial(
        pl.kernel,
        out_type=jax.ShapeDtypeStruct((2, na, d), jnp.float32),
        mesh=_sc_mesh(),
        scratch_types=[
            pltpu.VMEM((nb, CH), jnp.int32),
            pltpu.VMEM((nb, CH), jnp.int32),
            pltpu.VMEM((CH, d), jnp.float32),
            pltpu.VMEM_SHARED((na, d), jnp.float32),
            pltpu.SemaphoreType.DMA,
        ],
    )
    def msgpass(zs_hbm, edges_hbm, out_hbm, src_v, dst_v, rows_v, acc_sh, gsem):
        c = lax.axis_index("c")
        s = lax.axis_index("s")
        w = c * NS + s
        zeros = jnp.zeros((L,), jnp.float32)

        pltpu.sync_copy(edges_hbm.at[0, w], src_v)
        pltpu.sync_copy(edges_hbm.at[1, w], dst_v)

        def zrow(i, _):
            for k in range(d // L):
                rows_v[i, pl.ds(k * L, L)] = zeros
            return 0
        lax.fori_loop(0, CH, zrow, 0)
        for q in range(rpt // CH):
            pltpu.sync_copy(rows_v, acc_sh.at[pl.ds(s * rpt + q * CH, CH)])
        plsc.subcore_barrier()

        def edge_body(j, _):
            pltpu.async_copy(zs_hbm.at[src_v.at[j]], rows_v, gsem).wait()
            pltpu.sync_copy(rows_v, acc_sh.at[dst_v.at[j]], add=True)
            return 0
        lax.fori_loop(0, nb, edge_body, 0)

        plsc.subcore_barrier()
        pltpu.sync_copy(acc_sh.at[pl.ds(s * rpt, rpt)],
                        out_hbm.at[c, pl.ds(s * rpt, rpt)])

    return msgpass


def _tc_matmul_body(x_ref, d_ref, w_ref, o_ref):
    ns = lax.rsqrt(jnp.maximum(d_ref[...], 1.0))
    o_ref[...] = jnp.dot(x_ref[...] * ns, w_ref[...],
                         preferred_element_type=jnp.float32)


def _tc_combine_body(p_ref, d_ref, b_ref, o_ref):
    nd = lax.rsqrt(jnp.maximum(d_ref[...], 1.0))
    o_ref[...] = (p_ref[0] + p_ref[1]) * nd + b_ref[...]


def kernel(features, edge_index, W, b):
    n, d_in = features.shape
    d_out = W.shape[1]
    e = edge_index.shape[1]

    nw = NC * NS
    rpt = (-(-n // NS) + CH - 1) // CH * CH
    na = NS * rpt
    ept = -(-e // (nw * 4 * CH)) * 4 * CH
    nb = ept // CH
    e_pad = nw * ept

    src = edge_index[0]
    dst = edge_index[1]
    pad = jnp.full((e_pad - e,), n, dtype=jnp.int32)
    edges = jnp.stack([jnp.concatenate([src, pad]),
                       jnp.concatenate([dst, pad])]).reshape(2, nw, nb, CH)
    x_pad = jnp.zeros((na, d_in), features.dtype).at[:n].set(features)

    degs = _make_bincount(nw, nb, na)(edges)

    ds_col = degs[0].reshape(na, 1)
    rows_blk = 512
    grid = na // rows_blk
    zs = pl.pallas_call(
        _tc_matmul_body,
        grid=(grid,),
        in_specs=[
            pl.BlockSpec((rows_blk, d_in), lambda i: (i, 0)),
            pl.BlockSpec((rows_blk, 1), lambda i: (i, 0)),
            pl.BlockSpec((d_in, d_out), lambda i: (0, 0)),
        ],
        out_specs=pl.BlockSpec((rows_blk, d_out), lambda i: (i, 0)),
        out_shape=jax.ShapeDtypeStruct((na, d_out), jnp.float32),
    )(x_pad, ds_col, W)

    parts = _make_msgpass(nw, nb, na, d_out)(zs, edges)

    dd_col = degs[1].reshape(na, 1)
    b_row = b.reshape(1, d_out)
    out_blk = 1000
    out = pl.pallas_call(
        _tc_combine_body,
        grid=(n // out_blk,),
        in_specs=[
            pl.BlockSpec((2, out_blk, d_out), lambda i: (0, i, 0)),
            pl.BlockSpec((out_blk, 1), lambda i: (i, 0)),
            pl.BlockSpec((1, d_out), lambda i: (0, 0)),
        ],
        out_specs=pl.BlockSpec((out_blk, d_out), lambda i: (i, 0)),
        out_shape=jax.ShapeDtypeStruct((n, d_out), jnp.float32),
    )(parts, dd_col, b_row)

    return out


# speedup vs baseline: 1.5206x; 1.3431x over previous
import functools

import jax
import jax.numpy as jnp
from jax import lax
from jax.experimental import pallas as pl
from jax.experimental.pallas import tpu as pltpu
from jax.experimental.pallas import tpu_sc as plsc

NC = 2
NS = 16
L = 16
CH = 128


def _sc_mesh():
    return plsc.VectorSubcoreMesh(core_axis_name="c", subcore_axis_name="s")


def _make_bincount(nw, nb, na):
    rpt = na // NS
    bpt = nw // NS

    @functools.partial(
        pl.kernel,
        out_type=jax.ShapeDtypeStruct((2, na), jnp.float32),
        mesh=_sc_mesh(),
        compiler_params=pltpu.CompilerParams(needs_layout_passes=False),
        scratch_types=[
            pltpu.VMEM((nb, CH), jnp.int32),
            pltpu.VMEM((na,), jnp.float32),
            pltpu.VMEM((NS, rpt), jnp.float32),
            pltpu.VMEM((rpt,), jnp.float32),
            pltpu.VMEM_SHARED((NS, na), jnp.float32),
        ],
    )
    def bincount(edges_hbm, out_hbm, idx_v, hist_v, red_v, deg_v, shared_h):
        c = lax.axis_index("c")
        s = lax.axis_index("s")
        zeros = jnp.zeros((L,), jnp.float32)
        ones = jnp.ones((L,), jnp.float32)

        def zero_body(i, _):
            hist_v[pl.ds(i * L, L)] = zeros
            return 0
        lax.fori_loop(0, na // L, zero_body, 0)

        for bi in range(bpt):
            pltpu.sync_copy(edges_hbm.at[c, s * bpt + bi], idx_v)

            def acc_body(j, _):
                for k in range(CH // L):
                    idx16 = idx_v[j, pl.ds(k * L, L)]
                    plsc.addupdate_scatter(hist_v, [idx16], ones)
                return 0
            lax.fori_loop(0, nb, acc_body, 0)

        pltpu.sync_copy(hist_v, shared_h.at[s])
        plsc.subcore_barrier()

        for t in range(NS):
            pltpu.sync_copy(shared_h.at[t, pl.ds(s * rpt, rpt)], red_v.at[t])

        def red_body(i, _):
            v = red_v[0, pl.ds(i * L, L)]
            for t in range(1, NS):
                v = v + red_v[t, pl.ds(i * L, L)]
            deg_v[pl.ds(i * L, L)] = v
            return 0
        lax.fori_loop(0, rpt // L, red_body, 0)

        pltpu.sync_copy(deg_v, out_hbm.at[c, pl.ds(s * rpt, rpt)])

    return bincount


def _make_msgpass(nw, nb, na, d):
    rpt = na // NS

    @functools.partial(
        pl.kernel,
        out_type=jax.ShapeDtypeStruct((2, na, d), jnp.float32),
        mesh=_sc_mesh(),
        scratch_types=[
            pltpu.VMEM((nb, CH), jnp.int32),
            pltpu.VMEM((nb, CH), jnp.int32),
            pltpu.VMEM((CH, d), jnp.float32),
            pltpu.VMEM_SHARED((na, d), jnp.float32),
            pltpu.SemaphoreType.DMA,
        ],
    )
    def msgpass(zs_hbm, edges_hbm, out_hbm, src_v, dst_v, rows_v, acc_sh, gsem):
        c = lax.axis_index("c")
        s = lax.axis_index("s")
        w = c * NS + s
        zeros = jnp.zeros((L,), jnp.float32)

        pltpu.sync_copy(edges_hbm.at[0, w], src_v)
        pltpu.sync_copy(edges_hbm.at[1, w], dst_v)

        def zrow(i, _):
            for k in range(d // L):
                rows_v[i, pl.ds(k * L, L)] = zeros
            return 0
        lax.fori_loop(0, CH, zrow, 0)
        for q in range(rpt // CH):
            pltpu.sync_copy(rows_v, acc_sh.at[pl.ds(s * rpt + q * CH, CH)])
        plsc.subcore_barrier()

        def edge_body(j, _):
            pltpu.async_copy(zs_hbm.at[src_v.at[j]], rows_v, gsem).wait()
            pltpu.sync_copy(rows_v, acc_sh.at[dst_v.at[j]], add=True)
            return 0
        lax.fori_loop(0, nb, edge_body, 0)

        plsc.subcore_barrier()
        pltpu.sync_copy(acc_sh.at[pl.ds(s * rpt, rpt)],
                        out_hbm.at[c, pl.ds(s * rpt, rpt)])

    return msgpass


def _tc_matmul_body(x_ref, d_ref, w_ref, o_ref):
    ns = lax.rsqrt(jnp.maximum(d_ref[...], 1.0))
    o_ref[...] = jnp.dot(x_ref[...] * ns, w_ref[...],
                         preferred_element_type=jnp.float32)


def _tc_combine_body(p_ref, d_ref, b_ref, o_ref):
    nd = lax.rsqrt(jnp.maximum(d_ref[...], 1.0))
    o_ref[...] = (p_ref[0] + p_ref[1]) * nd + b_ref[...]


def kernel(features, edge_index, W, b):
    n, d_in = features.shape
    d_out = W.shape[1]
    e = edge_index.shape[1]

    nw = NC * NS
    rpt = (-(-n // NS) + CH - 1) // CH * CH
    na = NS * rpt
    ept = -(-e // (nw * CH)) * CH
    nb = ept // CH
    e_pad = nw * ept

    src = edge_index[0]
    dst = edge_index[1]
    pad = jnp.full((e_pad - e,), n, dtype=jnp.int32)
    edges = jnp.stack([jnp.concatenate([src, pad]),
                       jnp.concatenate([dst, pad])]).reshape(2, nw, nb, CH)
    x_pad = jnp.zeros((na, d_in), features.dtype).at[:n].set(features)

    degs = _make_bincount(nw, nb, na)(edges)

    ds_col = degs[0].reshape(na, 1)
    rows_blk = 512
    grid = na // rows_blk
    zs = pl.pallas_call(
        _tc_matmul_body,
        grid=(grid,),
        in_specs=[
            pl.BlockSpec((rows_blk, d_in), lambda i: (i, 0)),
            pl.BlockSpec((rows_blk, 1), lambda i: (i, 0)),
            pl.BlockSpec((d_in, d_out), lambda i: (0, 0)),
        ],
        out_specs=pl.BlockSpec((rows_blk, d_out), lambda i: (i, 0)),
        out_shape=jax.ShapeDtypeStruct((na, d_out), jnp.float32),
    )(x_pad, ds_col, W)

    parts = _make_msgpass(nw, nb, na, d_out)(zs, edges)

    dd_col = degs[1].reshape(na, 1)
    b_row = b.reshape(1, d_out)
    out_blk = 1000
    out = pl.pallas_call(
        _tc_combine_body,
        grid=(n // out_blk,),
        in_specs=[
            pl.BlockSpec((2, out_blk, d_out), lambda i: (0, i, 0)),
            pl.BlockSpec((out_blk, 1), lambda i: (i, 0)),
            pl.BlockSpec((1, d_out), lambda i: (0, 0)),
        ],
        out_specs=pl.BlockSpec((out_blk, d_out), lambda i: (i, 0)),
        out_shape=jax.ShapeDtypeStruct((n, d_out), jnp.float32),
    )(parts, dd_col, b_row)

    return out


# speedup vs baseline: 1.5271x; 1.0043x over previous
import functools

import jax
import jax.numpy as jnp
from jax import lax
from jax.experimental import pallas as pl
from jax.experimental.pallas import tpu as pltpu
from jax.experimental.pallas import tpu_sc as plsc

NC = 2
NS = 16
L = 16
CH = 128


def _sc_mesh():
    return plsc.VectorSubcoreMesh(core_axis_name="c", subcore_axis_name="s")


def _make_bincount(nw, nb, na):
    rpt = na // NS
    bpt = nw // NS

    @functools.partial(
        pl.kernel,
        out_type=jax.ShapeDtypeStruct((2, na), jnp.float32),
        mesh=_sc_mesh(),
        compiler_params=pltpu.CompilerParams(needs_layout_passes=False),
        scratch_types=[
            pltpu.VMEM((nb, CH), jnp.int32),
            pltpu.VMEM((na,), jnp.float32),
            pltpu.VMEM((NS, rpt), jnp.float32),
            pltpu.VMEM((rpt,), jnp.float32),
            pltpu.VMEM_SHARED((NS, na), jnp.float32),
        ],
    )
    def bincount(edges_hbm, out_hbm, idx_v, hist_v, red_v, deg_v, shared_h):
        c = lax.axis_index("c")
        s = lax.axis_index("s")
        zeros = jnp.zeros((L,), jnp.float32)
        ones = jnp.ones((L,), jnp.float32)

        def zero_body(i, _):
            hist_v[pl.ds(i * L, L)] = zeros
            return 0
        lax.fori_loop(0, na // L, zero_body, 0)

        for bi in range(bpt):
            pltpu.sync_copy(edges_hbm.at[c, s * bpt + bi], idx_v)

            def acc_body(j, _):
                for k in range(CH // L):
                    idx16 = idx_v[j, pl.ds(k * L, L)]
                    plsc.addupdate_scatter(hist_v, [idx16], ones)
                return 0
            lax.fori_loop(0, nb, acc_body, 0)

        pltpu.sync_copy(hist_v, shared_h.at[s])
        plsc.subcore_barrier()

        for t in range(NS):
            pltpu.sync_copy(shared_h.at[t, pl.ds(s * rpt, rpt)], red_v.at[t])

        def red_body(i, _):
            v = red_v[0, pl.ds(i * L, L)]
            for t in range(1, NS):
                v = v + red_v[t, pl.ds(i * L, L)]
            deg_v[pl.ds(i * L, L)] = v
            return 0
        lax.fori_loop(0, rpt // L, red_body, 0)

        pltpu.sync_copy(deg_v, out_hbm.at[c, pl.ds(s * rpt, rpt)])

    return bincount


def _make_msgpass(nw, nb, na, d):
    rpt = na // NS

    @functools.partial(
        pl.kernel,
        out_type=jax.ShapeDtypeStruct((2, na, d), jnp.float32),
        mesh=_sc_mesh(),
        scratch_types=[
            pltpu.VMEM((nb, CH), jnp.int32),
            pltpu.VMEM((nb, CH), jnp.int32),
            pltpu.VMEM((CH, d), jnp.float32),
            pltpu.VMEM_SHARED((na, d), jnp.float32),
            pltpu.SemaphoreType.DMA,
        ],
    )
    def msgpass(zs_hbm, edges_hbm, out_hbm, src_v, dst_v, rows_v, acc_sh, gsem):
        c = lax.axis_index("c")
        s = lax.axis_index("s")
        w = c * NS + s
        zeros = jnp.zeros((L,), jnp.float32)

        pltpu.sync_copy(edges_hbm.at[0, w], src_v)
        pltpu.sync_copy(edges_hbm.at[1, w], dst_v)

        def zrow(i, _):
            for k in range(d // L):
                rows_v[i, pl.ds(k * L, L)] = zeros
            return 0
        lax.fori_loop(0, CH, zrow, 0)
        for q in range(rpt // CH):
            pltpu.sync_copy(rows_v, acc_sh.at[pl.ds(s * rpt + q * CH, CH)])
        plsc.subcore_barrier()

        def edge_body(j, _):
            pltpu.async_copy(zs_hbm.at[src_v.at[j]], rows_v, gsem).wait()
            pltpu.sync_copy(rows_v, acc_sh.at[dst_v.at[j]], add=True)
            return 0
        lax.fori_loop(0, nb, edge_body, 0)

        plsc.subcore_barrier()
        pltpu.sync_copy(acc_sh.at[pl.ds(s * rpt, rpt)],
                        out_hbm.at[c, pl.ds(s * rpt, rpt)])

    return msgpass


def _tc_matmul_body(x_ref, w_ref, o_ref):
    o_ref[...] = jnp.dot(x_ref[...], w_ref[...],
                         preferred_element_type=jnp.float32)


def _tc_scale_body(y_ref, d_ref, o_ref):
    o_ref[...] = y_ref[...] * lax.rsqrt(jnp.maximum(d_ref[...], 1.0))


def _tc_combine_body(p_ref, d_ref, b_ref, o_ref):
    nd = lax.rsqrt(jnp.maximum(d_ref[...], 1.0))
    o_ref[...] = (p_ref[0] + p_ref[1]) * nd + b_ref[...]


def kernel(features, edge_index, W, b):
    n, d_in = features.shape
    d_out = W.shape[1]
    e = edge_index.shape[1]

    nw = NC * NS
    rpt = (-(-n // NS) + CH - 1) // CH * CH
    na = NS * rpt
    ept = -(-e // (nw * CH)) * CH
    nb = ept // CH
    e_pad = nw * ept

    pad2 = jnp.full((2, e_pad - e), n, jnp.int32)
    edges = jnp.concatenate([edge_index, pad2], axis=1).reshape(2, nw, nb, CH)

    degs = _make_bincount(nw, nb, na)(edges)
    rows_blk = 2000
    y = pl.pallas_call(
        _tc_matmul_body,
        grid=(n // rows_blk,),
        in_specs=[
            pl.BlockSpec((rows_blk, d_in), lambda i: (i, 0)),
            pl.BlockSpec((d_in, d_out), lambda i: (0, 0)),
        ],
        out_specs=pl.BlockSpec((rows_blk, d_out), lambda i: (i, 0)),
        out_shape=jax.ShapeDtypeStruct((n, d_out), jnp.float32),
    )(features, W)

    ds_col = degs[0, :n].reshape(n, 1)
    zs = pl.pallas_call(
        _tc_scale_body,
        grid=(n // rows_blk,),
        in_specs=[
            pl.BlockSpec((rows_blk, d_out), lambda i: (i, 0)),
            pl.BlockSpec((rows_blk, 1), lambda i: (i, 0)),
        ],
        out_specs=pl.BlockSpec((rows_blk, d_out), lambda i: (i, 0)),
        out_shape=jax.ShapeDtypeStruct((na, d_out), jnp.float32),
    )(y, ds_col)

    parts = _make_msgpass(nw, nb, na, d_out)(zs, edges)

    dd_col = degs[1].reshape(na, 1)
    b_row = b.reshape(1, d_out)
    out_blk = 1000
    out = pl.pallas_call(
        _tc_combine_body,
        grid=(n // out_blk,),
        in_specs=[
            pl.BlockSpec((2, out_blk, d_out), lambda i: (0, i, 0)),
            pl.BlockSpec((out_blk, 1), lambda i: (i, 0)),
            pl.BlockSpec((1, d_out), lambda i: (0, 0)),
        ],
        out_specs=pl.BlockSpec((out_blk, d_out), lambda i: (i, 0)),
        out_shape=jax.ShapeDtypeStruct((n, d_out), jnp.float32),
    )(parts, dd_col, b_row)

    return out
